# baseline (speedup 1.0000x reference)
import functools

import jax
import jax.numpy as jnp
from jax import lax
from jax.experimental import pallas as pl
from jax.experimental.pallas import tpu as pltpu
from jax.experimental.pallas import tpu_sc as plsc

N = 320000
N_IN = 128
N_HID = 32
SEGS = 8000
N_CT = 16
NBAGS = SEGS // N_CT

EW = 16
BLK_A = 4000

NW = 32
RW = N // NW
SUB = 125
NSUB = 8
BIG = SUB * NSUB
NBIG = RW // BIG
BROWS = N // SUB
ZROWS = SEGS // 8


def _stage_a(x_ref, w1_ref, b1_ref, wc_ref, bc_ref, he_ref, e_ref):
    x = x_ref[...]
    h = jnp.maximum(
        jnp.dot(x, w1_ref[...], preferred_element_type=jnp.float32) + b1_ref[...],
        0.0,
    )
    s = jnp.sum(h * wc_ref[...], axis=1, keepdims=True) + bc_ref[...]
    e = jnp.exp(s)
    he_ref[...] = h * e
    e_ref[...] = e * jnp.ones((1, EW), jnp.float32)


def _stage_b(he_hbm, e_hbm, b_hbm, u_hbm, d_hbm, hbuf, ebuf, ibuf, uacc, dacc,
             lsem, ssem):
    cid = lax.axis_index("c")
    sid = lax.axis_index("s")
    wid = sid * 2 + cid

    z = jnp.zeros((16,), jnp.float32)

    def zrow(i, carry):
        hbuf[0, i, pl.ds(0, 16)] = z
        hbuf[0, i, pl.ds(16, 16)] = z
        ebuf[0, i, pl.ds(0, 16)] = z
        return carry

    @pl.when(sid < 8)
    def _zero():
        lax.fori_loop(0, ZROWS, zrow, 0)
        pltpu.sync_copy(hbuf.at[0], uacc.at[pl.ds(sid * ZROWS, ZROWS)])
        pltpu.sync_copy(ebuf.at[0], dacc.at[pl.ds(sid * ZROWS, ZROWS)])

    plsc.subcore_barrier()

    rowbase = wid * RW
    ibase = wid * (RW // SUB)

    def start_load(bi, b):
        hh = pltpu.async_copy(
            he_hbm.at[pl.ds(rowbase + bi * BIG, BIG)], hbuf.at[b], lsem
        )
        he_ = pltpu.async_copy(
            e_hbm.at[pl.ds(rowbase + bi * BIG, BIG)], ebuf.at[b], lsem
        )
        hi = pltpu.async_copy(
            b_hbm.at[pl.ds(ibase + bi * NSUB, NSUB)], ibuf.at[b], lsem
        )
        return hh, he_, hi

    pend = start_load(0, 0)
    for bi in range(NBIG):
        b = bi % 2
        h0, h1, h2 = pend
        if bi + 1 < NBIG:
            pend = start_load(bi + 1, (bi + 1) % 2)
        h0.wait()
        h1.wait()
        h2.wait()
        scs = []
        for j in range(NSUB):
            scs.append(
                pltpu.async_copy(
                    hbuf.at[b, pl.ds(j * SUB, SUB)],
                    uacc.at[ibuf.at[b, j]],
                    ssem,
                    add=True,
                )
            )
            scs.append(
                pltpu.async_copy(
                    ebuf.at[b, pl.ds(j * SUB, SUB)],
                    dacc.at[ibuf.at[b, j]],
                    ssem,
                    add=True,
                )
            )
        for h in scs:
            h.wait()

    plsc.subcore_barrier()

    @pl.when(sid < 8)
    def _flush():
        pltpu.sync_copy(
            uacc.at[pl.ds(sid * ZROWS, ZROWS)],
            u_hbm.at[cid, pl.ds(sid * ZROWS, ZROWS)],
        )
        pltpu.sync_copy(
            dacc.at[pl.ds(sid * ZROWS, ZROWS)],
            d_hbm.at[cid, pl.ds(sid * ZROWS, ZROWS)],
        )


def _stage_c(u_ref, d_ref, wct_ref, bct_ref, wo_ref, bo_ref, out_ref):
    u = u_ref[...]
    d = d_ref[...]
    uc = u[0] + u[1]
    dc = d[0, :, :, :1] + d[1, :, :, :1]
    pooled = uc / (dc + 1e-16)
    t = jnp.sum(pooled * wct_ref[...], axis=-1, keepdims=True) + bct_ref[...]
    m = jnp.max(t, axis=1, keepdims=True)
    ee = jnp.exp(t - m)
    dd = jnp.sum(ee, axis=1, keepdims=True)
    xs = jnp.sum(pooled * (ee / dd), axis=1)
    out_ref[...] = jnp.sum(xs * wo_ref[...], axis=-1, keepdims=True) + bo_ref[...]


def kernel(X, batch, ct_size, n_ct, W1, b1, Wc, bc, Wct, bct, Wo, bo):
    he, e16 = pl.pallas_call(
        _stage_a,
        grid=(N // BLK_A,),
        in_specs=[
            pl.BlockSpec((BLK_A, N_IN), lambda i: (i, 0)),
            pl.BlockSpec((N_IN, N_HID), lambda i: (0, 0)),
            pl.BlockSpec((1, N_HID), lambda i: (0, 0)),
            pl.BlockSpec((1, N_HID), lambda i: (0, 0)),
            pl.BlockSpec((1, 1), lambda i: (0, 0)),
        ],
        out_specs=[
            pl.BlockSpec((BLK_A, N_HID), lambda i: (i, 0)),
            pl.BlockSpec((BLK_A, EW), lambda i: (i, 0)),
        ],
        out_shape=[
            jax.ShapeDtypeStruct((N, N_HID), jnp.float32),
            jax.ShapeDtypeStruct((N, EW), jnp.float32),
        ],
    )(
        X,
        W1,
        b1.reshape(1, N_HID),
        Wc[:, 0].reshape(1, N_HID),
        bc.reshape(1, 1),
    )

    mesh = plsc.VectorSubcoreMesh(core_axis_name="c", subcore_axis_name="s")
    u2, d2 = pl.kernel(
        _stage_b,
        mesh=mesh,
        compiler_params=pltpu.CompilerParams(use_tc_tiling_on_sc=False),
        out_type=[
            jax.ShapeDtypeStruct((2, SEGS, N_HID), jnp.float32),
            jax.ShapeDtypeStruct((2, SEGS, EW), jnp.float32),
        ],
        scratch_types=[
            pltpu.VMEM((2, BIG, N_HID), jnp.float32),
            pltpu.VMEM((2, BIG, EW), jnp.float32),
            pltpu.VMEM((2, NSUB, SUB), jnp.int32),
            pltpu.VMEM_SHARED((SEGS, N_HID), jnp.float32),
            pltpu.VMEM_SHARED((SEGS, EW), jnp.float32),
            pltpu.SemaphoreType.DMA,
            pltpu.SemaphoreType.DMA,
        ],
    )(he, e16, batch.reshape(BROWS, SUB))

    out = pl.pallas_call(
        _stage_c,
        out_shape=jax.ShapeDtypeStruct((NBAGS, 1), jnp.float32),
    )(
        u2.reshape(2, NBAGS, N_CT, N_HID),
        d2.reshape(2, NBAGS, N_CT, EW),
        Wct[:, 0].reshape(1, 1, N_HID),
        bct.reshape(1, 1, 1),
        Wo[:, 0].reshape(1, N_HID),
        bo.reshape(1, 1),
    )
    return out


# speedup vs baseline: 1.1937x; 1.1937x over previous
import functools

import jax
import jax.numpy as jnp
from jax import lax
from jax.experimental import pallas as pl
from jax.experimental.pallas import tpu as pltpu
from jax.experimental.pallas import tpu_sc as plsc

N = 320000
N_IN = 128
N_HID = 32
SEGS = 8000
N_CT = 16
NBAGS = SEGS // N_CT

GW = 48
BLK_A = 4000

NW = 32
RW = N // NW
SUB = 125
NSUB = 8
BIG = SUB * NSUB
NBIG = RW // BIG
BROWS = N // SUB
ZROWS = SEGS // 8


def _stage_a(x_ref, w1_ref, b1_ref, wc_ref, bc_ref, g_ref):
    x = x_ref[...]
    h = jnp.maximum(
        jnp.dot(x, w1_ref[...], preferred_element_type=jnp.float32) + b1_ref[...],
        0.0,
    )
    s = jnp.sum(h * wc_ref[...], axis=1, keepdims=True) + bc_ref[...]
    e = jnp.exp(s)
    g_ref[...] = jnp.concatenate(
        [h * e, e, jnp.zeros((BLK_A, GW - N_HID - 1), jnp.float32)], axis=1
    )


def _stage_b(g_hbm, b_hbm, out_hbm, gbuf, ibuf, uacc, lsem, ssem):
    cid = lax.axis_index("c")
    sid = lax.axis_index("s")
    wid = sid * 2 + cid

    z = jnp.zeros((16,), jnp.float32)

    def zrow(i, carry):
        gbuf[0, i, pl.ds(0, 16)] = z
        gbuf[0, i, pl.ds(16, 16)] = z
        gbuf[0, i, pl.ds(32, 16)] = z
        return carry

    @pl.when(sid < 8)
    def _zero():
        lax.fori_loop(0, ZROWS, zrow, 0)
        pltpu.sync_copy(gbuf.at[0], uacc.at[pl.ds(sid * ZROWS, ZROWS)])

    plsc.subcore_barrier()

    rowbase = wid * RW
    ibase = wid * (RW // SUB)

    def start_load(bi, b):
        hg = pltpu.async_copy(
            g_hbm.at[pl.ds(rowbase + bi * BIG, BIG)], gbuf.at[b], lsem
        )
        hi = pltpu.async_copy(
            b_hbm.at[pl.ds(ibase + bi * NSUB, NSUB)], ibuf.at[b], lsem
        )
        return hg, hi

    pend = start_load(0, 0)
    for bi in range(NBIG):
        b = bi % 2
        hg, hi = pend
        if bi + 1 < NBIG:
            pend = start_load(bi + 1, (bi + 1) % 2)
        hg.wait()
        hi.wait()
        scs = [
            pltpu.async_copy(
                gbuf.at[b, pl.ds(j * SUB, SUB)],
                uacc.at[ibuf.at[b, j]],
                ssem,
                add=True,
            )
            for j in range(NSUB)
        ]
        for h in scs:
            h.wait()

    plsc.subcore_barrier()

    @pl.when(sid < 8)
    def _flush():
        pltpu.sync_copy(
            uacc.at[pl.ds(sid * ZROWS, ZROWS)],
            out_hbm.at[cid, pl.ds(sid * ZROWS, ZROWS)],
        )


def _stage_c(u_ref, wct_ref, bct_ref, wo_ref, bo_ref, out_ref):
    u = u_ref[...]
    uc = u[0] + u[1]
    pooled = uc[:, :, :N_HID] / (uc[:, :, N_HID:N_HID + 1] + 1e-16)
    t = jnp.sum(pooled * wct_ref[...], axis=-1, keepdims=True) + bct_ref[...]
    m = jnp.max(t, axis=1, keepdims=True)
    ee = jnp.exp(t - m)
    dd = jnp.sum(ee, axis=1, keepdims=True)
    xs = jnp.sum(pooled * (ee / dd), axis=1)
    out_ref[...] = jnp.sum(xs * wo_ref[...], axis=-1, keepdims=True) + bo_ref[...]


def kernel(X, batch, ct_size, n_ct, W1, b1, Wc, bc, Wct, bct, Wo, bo):
    g = pl.pallas_call(
        _stage_a,
        grid=(N // BLK_A,),
        in_specs=[
            pl.BlockSpec((BLK_A, N_IN), lambda i: (i, 0)),
            pl.BlockSpec((N_IN, N_HID), lambda i: (0, 0)),
            pl.BlockSpec((1, N_HID), lambda i: (0, 0)),
            pl.BlockSpec((1, N_HID), lambda i: (0, 0)),
            pl.BlockSpec((1, 1), lambda i: (0, 0)),
        ],
        out_specs=pl.BlockSpec((BLK_A, GW), lambda i: (i, 0)),
        out_shape=jax.ShapeDtypeStruct((N, GW), jnp.float32),
    )(
        X,
        W1,
        b1.reshape(1, N_HID),
        Wc[:, 0].reshape(1, N_HID),
        bc.reshape(1, 1),
    )

    mesh = plsc.VectorSubcoreMesh(core_axis_name="c", subcore_axis_name="s")
    u2 = pl.kernel(
        _stage_b,
        mesh=mesh,
        compiler_params=pltpu.CompilerParams(use_tc_tiling_on_sc=False),
        out_type=jax.ShapeDtypeStruct((2, SEGS, GW), jnp.float32),
        scratch_types=[
            pltpu.VMEM((2, BIG, GW), jnp.float32),
            pltpu.VMEM((2, NSUB, SUB), jnp.int32),
            pltpu.VMEM_SHARED((SEGS, GW), jnp.float32),
            pltpu.SemaphoreType.DMA,
            pltpu.SemaphoreType.DMA,
        ],
    )(g, batch.reshape(BROWS, SUB))

    out = pl.pallas_call(
        _stage_c,
        out_shape=jax.ShapeDtypeStruct((NBAGS, 1), jnp.float32),
    )(
        u2.reshape(2, NBAGS, N_CT, GW),
        Wct[:, 0].reshape(1, 1, N_HID),
        bct.reshape(1, 1, 1),
        Wo[:, 0].reshape(1, N_HID),
        bo.reshape(1, 1),
    )
    return out


# speedup vs baseline: 1.2206x; 1.0225x over previous
import functools

import jax
import jax.numpy as jnp
from jax import lax
from jax.experimental import pallas as pl
from jax.experimental.pallas import tpu as pltpu
from jax.experimental.pallas import tpu_sc as plsc

N = 320000
N_IN = 128
N_HID = 32
SEGS = 8000
N_CT = 16
NBAGS = SEGS // N_CT

GW = 48
BLK_A = 4000

NW = 32
RW = N // NW
SUB = 125
NSUB = 8
BIG = SUB * NSUB
NBIG = RW // BIG
BROWS = N // SUB
ZROWS = SEGS // 8


def _stage_a(x_ref, w1_ref, b1_ref, wc_ref, bc_ref, g_ref):
    x = x_ref[...]
    h = jnp.maximum(
        jnp.dot(x, w1_ref[...], preferred_element_type=jnp.float32) + b1_ref[...],
        0.0,
    )
    s = jnp.sum(h * wc_ref[...], axis=1, keepdims=True) + bc_ref[...]
    e = jnp.exp(s)
    g_ref[...] = jnp.concatenate(
        [h * e, e, jnp.zeros((BLK_A, GW - N_HID - 1), jnp.float32)], axis=1
    )


def _stage_b(nbig, ib0, g_hbm, b_hbm, out_hbm, gbuf, ibuf, uacc, lsem, ssem):
    rw = nbig * BIG
    cid = lax.axis_index("c")
    sid = lax.axis_index("s")
    wid = sid * 2 + cid

    z = jnp.zeros((16,), jnp.float32)

    def zrow(i, carry):
        gbuf[0, i, pl.ds(0, 16)] = z
        gbuf[0, i, pl.ds(16, 16)] = z
        gbuf[0, i, pl.ds(32, 16)] = z
        return carry

    @pl.when(sid < 8)
    def _zero():
        lax.fori_loop(0, ZROWS, zrow, 0)
        pltpu.sync_copy(gbuf.at[0], uacc.at[pl.ds(sid * ZROWS, ZROWS)])

    plsc.subcore_barrier()

    rowbase = wid * rw
    ibase = ib0 + wid * (rw // SUB)

    def start_load(bi, b):
        hg = pltpu.async_copy(
            g_hbm.at[pl.ds(rowbase + bi * BIG, BIG)], gbuf.at[b], lsem
        )
        hi = pltpu.async_copy(
            b_hbm.at[pl.ds(ibase + bi * NSUB, NSUB)], ibuf.at[b], lsem
        )
        return hg, hi

    pend = start_load(0, 0)
    for bi in range(nbig):
        b = bi % 2
        hg, hi = pend
        if bi + 1 < nbig:
            pend = start_load(bi + 1, (bi + 1) % 2)
        hg.wait()
        hi.wait()
        scs = [
            pltpu.async_copy(
                gbuf.at[b, pl.ds(j * SUB, SUB)],
                uacc.at[ibuf.at[b, j]],
                ssem,
                add=True,
            )
            for j in range(NSUB)
        ]
        for h in scs:
            h.wait()

    plsc.subcore_barrier()

    @pl.when(sid < 8)
    def _flush():
        pltpu.sync_copy(
            uacc.at[pl.ds(sid * ZROWS, ZROWS)],
            out_hbm.at[cid, pl.ds(sid * ZROWS, ZROWS)],
        )


def _stage_c(ua_ref, ub_ref, wct_ref, bct_ref, wo_ref, bo_ref, out_ref):
    ua = ua_ref[...]
    ub = ub_ref[...]
    uc = (ua[0] + ua[1]) + (ub[0] + ub[1])
    pooled = uc[:, :, :N_HID] / (uc[:, :, N_HID:N_HID + 1] + 1e-16)
    t = jnp.sum(pooled * wct_ref[...], axis=-1, keepdims=True) + bct_ref[...]
    m = jnp.max(t, axis=1, keepdims=True)
    ee = jnp.exp(t - m)
    dd = jnp.sum(ee, axis=1, keepdims=True)
    xs = jnp.sum(pooled * (ee / dd), axis=1)
    out_ref[...] = jnp.sum(xs * wo_ref[...], axis=-1, keepdims=True) + bo_ref[...]


def kernel(X, batch, ct_size, n_ct, W1, b1, Wc, bc, Wct, bct, Wo, bo):
    NH = N // 2
    nbig_h = NH // NW // BIG

    def stage_a_half(block_off):
        return pl.pallas_call(
            _stage_a,
            grid=(NH // BLK_A,),
            in_specs=[
                pl.BlockSpec((BLK_A, N_IN), lambda i: (i + block_off, 0)),
                pl.BlockSpec((N_IN, N_HID), lambda i: (0, 0)),
                pl.BlockSpec((1, N_HID), lambda i: (0, 0)),
                pl.BlockSpec((1, N_HID), lambda i: (0, 0)),
                pl.BlockSpec((1, 1), lambda i: (0, 0)),
            ],
            out_specs=pl.BlockSpec((BLK_A, GW), lambda i: (i, 0)),
            out_shape=jax.ShapeDtypeStruct((NH, GW), jnp.float32),
        )(
            X,
            W1,
            b1.reshape(1, N_HID),
            Wc[:, 0].reshape(1, N_HID),
            bc.reshape(1, 1),
        )

    mesh = plsc.VectorSubcoreMesh(core_axis_name="c", subcore_axis_name="s")
    batch2d = batch.reshape(BROWS, SUB)

    def stage_b_half(g_half, ib0):
        return pl.kernel(
            functools.partial(_stage_b, nbig_h, ib0),
            mesh=mesh,
            compiler_params=pltpu.CompilerParams(use_tc_tiling_on_sc=False),
            out_type=jax.ShapeDtypeStruct((2, SEGS, GW), jnp.float32),
            scratch_types=[
                pltpu.VMEM((2, BIG, GW), jnp.float32),
                pltpu.VMEM((2, NSUB, SUB), jnp.int32),
                pltpu.VMEM_SHARED((SEGS, GW), jnp.float32),
                pltpu.SemaphoreType.DMA,
                pltpu.SemaphoreType.DMA,
            ],
        )(g_half, batch2d)

    g0 = stage_a_half(0)
    u2a = stage_b_half(g0, 0)
    g1 = stage_a_half(NH // BLK_A)
    u2b = stage_b_half(g1, (NH // SUB))

    out = pl.pallas_call(
        _stage_c,
        out_shape=jax.ShapeDtypeStruct((NBAGS, 1), jnp.float32),
    )(
        u2a.reshape(2, NBAGS, N_CT, GW),
        u2b.reshape(2, NBAGS, N_CT, GW),
        Wct[:, 0].reshape(1, 1, N_HID),
        bct.reshape(1, 1, 1),
        Wo[:, 0].reshape(1, N_HID),
        bo.reshape(1, 1),
    )
    return out
